# Initial kernel scaffold; baseline (speedup 1.0000x reference)
import jax
import jax.numpy as jnp
from jax.experimental import pallas as pl


def kernel(features_0, indices_0, features_1, indices_1, features_2, indices_2, features_3, indices_3, params):
    raise NotImplementedError("write your pallas kernel here")


# speedup vs baseline: 2.8193x; 2.8193x over previous
import functools

import jax
import jax.numpy as jnp
from jax import lax
from jax.experimental import pallas as pl
from jax.experimental.pallas import tpu as pltpu
from jax.experimental.pallas import tpu_sc as plsc

_DM = 512
_DFF = 1024
_DCHL = [64, 128, 256, 256]
_SHAPES = [(8, 200, 176), (4, 100, 88), (2, 50, 44), (2, 25, 22)]
_NVOX = [20000, 10000, 3000, 1000]
_NPTS = 16

_SC_CQ = [64, 32, 16, 16]
_SC_NITER = [10, 10, 6, 2]
_NTILES = 32


def _f32(x):
    return x.astype(jnp.float32)


def _make_stage_a(lvl):
    c = _DCHL[lvl]
    D, H, W = _SHAPES[lvl]
    n = _NVOX[lvl]
    gs = _DM // c
    G = c
    bn = 1000
    HW = H * W
    Dm1, Hm1, Wm1 = float(D - 1), float(H - 1), float(W - 1)

    def body(feat, idx, inWt, inb, gnw, gnb, qWt, qb, vWt, vb, offWt, offb,
             awWt, awb, q_o, v_o, sflat_o, w2_o, vflat_o):
        x = feat[...]
        src = jnp.dot(x, inWt[...], preferred_element_type=jnp.float32) + inb[...]
        r2 = lax.broadcasted_iota(jnp.int32, (_DM, G), 0)
        c2 = lax.broadcasted_iota(jnp.int32, (_DM, G), 1)
        M = (r2 // gs == c2).astype(jnp.float32)
        r3 = lax.broadcasted_iota(jnp.int32, (G, _DM), 0)
        c3 = lax.broadcasted_iota(jnp.int32, (G, _DM), 1)
        MT = (c3 // gs == r3).astype(jnp.float32)
        s1 = jnp.dot(src, M, precision=lax.Precision.HIGHEST, preferred_element_type=jnp.float32)
        meanfull = jnp.dot(s1 * (1.0 / gs), MT, precision=lax.Precision.HIGHEST, preferred_element_type=jnp.float32)
        dev = src - meanfull
        s2 = jnp.dot(dev * dev, M, precision=lax.Precision.HIGHEST, preferred_element_type=jnp.float32)
        varfull = jnp.dot(s2 * (1.0 / gs), MT, precision=lax.Precision.HIGHEST, preferred_element_type=jnp.float32)
        srcgn = dev * lax.rsqrt(varfull + 1e-5) * gnw[...] + gnb[...]

        q = jnp.dot(srcgn, qWt[...], preferred_element_type=jnp.float32) + qb[...]
        v = jnp.dot(srcgn, vWt[...], preferred_element_type=jnp.float32) + vb[...]
        offl = jnp.dot(q, offWt[...], preferred_element_type=jnp.float32) + offb[...]
        awl = jnp.dot(q, awWt[...], preferred_element_type=jnp.float32) + awb[...]

        mx = jnp.max(awl, axis=-1, keepdims=True)
        e = jnp.exp(awl - mx)
        r4 = lax.broadcasted_iota(jnp.int32, (_NPTS, _NPTS), 0)
        c4 = lax.broadcasted_iota(jnp.int32, (_NPTS, _NPTS), 1)
        B16 = (r4 // 4 == c4 // 4).astype(jnp.float32)
        den = jnp.dot(e, B16, precision=lax.Precision.HIGHEST, preferred_element_type=jnp.float32)
        aw = e / den

        rj = lax.broadcasted_iota(jnp.int32, (3 * _NPTS, _NPTS), 0)
        pj = lax.broadcasted_iota(jnp.int32, (3 * _NPTS, _NPTS), 1)
        S0 = (rj == 3 * pj).astype(jnp.float32)
        S1 = (rj == 3 * pj + 1).astype(jnp.float32)
        S2 = (rj == 3 * pj + 2).astype(jnp.float32)
        fz = jnp.dot(offl, S0, precision=lax.Precision.HIGHEST, preferred_element_type=jnp.float32)
        fy = jnp.dot(offl, S1, precision=lax.Precision.HIGHEST, preferred_element_type=jnp.float32)
        fx = jnp.dot(offl, S2, precision=lax.Precision.HIGHEST, preferred_element_type=jnp.float32)

        idxv = idx[...]
        dd = _f32(idxv[:, 1:2])
        hh = _f32(idxv[:, 2:3])
        ww = _f32(idxv[:, 3:4])

        def samp(f, coord, Nm1):
            loc = f / Nm1 + coord / Nm1
            grid = 2.0 * loc - 1.0
            return (grid + 1.0) / 2.0 * Nm1

        gz = samp(fz, dd, Dm1)
        gy = samp(fy, hh, Hm1)
        gx = samp(fx, ww, Wm1)
        ix = jnp.round(gx).astype(jnp.int32)
        iy = jnp.round(gy).astype(jnp.int32)
        iz = jnp.round(gz).astype(jnp.int32)
        valid = ((ix >= 0) & (ix < W) & (iy >= 0) & (iy < H)
                 & (iz >= 0) & (iz < D))
        ixc = jnp.clip(ix, 0, W - 1)
        iyc = jnp.clip(iy, 0, H - 1)
        izc = jnp.clip(iz, 0, D - 1)

        q_o[...] = q
        v_o[...] = v
        sflat_o[...] = izc * HW + iyc * W + ixc
        w2_o[...] = aw * valid.astype(jnp.float32)
        vf = idxv[:, 1:2] * HW + idxv[:, 2:3] * W + idxv[:, 3:4]
        vflat_o[...] = jnp.broadcast_to(vf, (bn, _NPTS))

    grid = (n // bn,)

    def blk(i):
        return (i, 0)

    def wblk(i):
        return (0, 0)

    in_specs = [
        pl.BlockSpec((bn, c), blk),
        pl.BlockSpec((bn, 4), blk),
        pl.BlockSpec((c, _DM), wblk),
        pl.BlockSpec((1, _DM), wblk),
        pl.BlockSpec((1, _DM), wblk),
        pl.BlockSpec((1, _DM), wblk),
        pl.BlockSpec((_DM, c), wblk),
        pl.BlockSpec((1, c), wblk),
        pl.BlockSpec((_DM, c), wblk),
        pl.BlockSpec((1, c), wblk),
        pl.BlockSpec((c, 3 * _NPTS), wblk),
        pl.BlockSpec((1, 3 * _NPTS), wblk),
        pl.BlockSpec((c, _NPTS), wblk),
        pl.BlockSpec((1, _NPTS), wblk),
    ]
    out_specs = [
        pl.BlockSpec((bn, c), blk),
        pl.BlockSpec((bn, c), blk),
        pl.BlockSpec((bn, _NPTS), blk),
        pl.BlockSpec((bn, _NPTS), blk),
        pl.BlockSpec((bn, _NPTS), blk),
    ]
    out_shape = [
        jax.ShapeDtypeStruct((n, c), jnp.float32),
        jax.ShapeDtypeStruct((n, c), jnp.float32),
        jax.ShapeDtypeStruct((n, _NPTS), jnp.int32),
        jax.ShapeDtypeStruct((n, _NPTS), jnp.float32),
        jax.ShapeDtypeStruct((n, _NPTS), jnp.int32),
    ]
    return pl.pallas_call(body, grid=grid, in_specs=in_specs,
                          out_specs=out_specs, out_shape=out_shape)


def _make_sc(lvl):
    c = _DCHL[lvl]
    D, H, W = _SHAPES[lvl]
    n = _NVOX[lvl]
    DHW = D * H * W
    CQ = _SC_CQ[lvl]
    NITER = _SC_NITER[lvl]
    qpt = CQ * NITER
    Np = _NTILES * qpt
    P = CQ * _NPTS
    epv = 16 * ((n + 16 * _NTILES - 1) // (16 * _NTILES))
    Nv = _NTILES * epv
    LUTN = DHW + 8
    NG = c // 16

    mesh = plsc.VectorSubcoreMesh(core_axis_name="cc", subcore_axis_name="ss",
                                  num_cores=2, num_subcores=16)

    @functools.partial(
        pl.kernel,
        out_type=jax.ShapeDtypeStruct((LUTN,), jnp.int32),
        mesh=mesh,
        compiler_params=pltpu.CompilerParams(use_tc_tiling_on_sc=False),
        scratch_types=[
            pltpu.VMEM((epv,), jnp.int32),
            pltpu.VMEM((epv,), jnp.int32),
            pltpu.SemaphoreType.DMA,
        ],
    )
    def sc_scatter(vflat, lut, idx_v, val_v, sem):
        ci = lax.axis_index("cc")
        si = lax.axis_index("ss")
        wid = si * 2 + ci
        base = wid * epv
        pltpu.sync_copy(vflat.at[pl.ds(base, epv)], idx_v)

        def fill(j, _):
            val_v[pl.ds(j * 16, 16)] = (base + j * 16
                                        + lax.iota(jnp.int32, 16))
            return 0

        lax.fori_loop(0, epv // 16, fill, 0)
        pltpu.async_copy(val_v, lut.at[idx_v], sem).wait()

    @functools.partial(
        pl.kernel,
        out_type=jax.ShapeDtypeStruct((Np, c), jnp.float32),
        mesh=mesh,
        compiler_params=pltpu.CompilerParams(use_tc_tiling_on_sc=False),
        scratch_types=[
            pltpu.VMEM((P,), jnp.int32),
            pltpu.VMEM((P,), jnp.float32),
            pltpu.VMEM((P,), jnp.int32),
            pltpu.VMEM((P,), jnp.int32),
            pltpu.VMEM((P, c), jnp.float32),
            pltpu.VMEM((CQ, c), jnp.float32),
            pltpu.SemaphoreType.DMA,
        ],
    )
    def sc_gather(vflat, sflat, w, v, lut, att, sf_v, w_v, r_v,
                  key_v, rows_v, out_v, sem):
        ci = lax.axis_index("cc")
        si = lax.axis_index("ss")
        wid = si * 2 + ci

        def chunk(it, _):
            qbase = wid * qpt + it * CQ
            pbase = qbase * _NPTS
            pltpu.sync_copy(sflat.at[pl.ds(pbase, P)], sf_v)
            pltpu.sync_copy(w.at[pl.ds(pbase, P)], w_v)
            pltpu.async_copy(lut.at[sf_v], r_v, sem).wait()

            def clampj(j, _):
                rr = r_v[pl.ds(j * 16, 16)]
                r_v[pl.ds(j * 16, 16)] = jnp.clip(rr, 0, n - 1)
                return 0

            lax.fori_loop(0, P // 16, clampj, 0)
            pltpu.async_copy(vflat.at[r_v], key_v, sem).wait()

            def maskj(j, _):
                sel = key_v[pl.ds(j * 16, 16)] == sf_v[pl.ds(j * 16, 16)]
                w_v[pl.ds(j * 16, 16)] = jnp.where(
                    sel, w_v[pl.ds(j * 16, 16)], 0.0)
                return 0

            lax.fori_loop(0, P // 16, maskj, 0)
            pltpu.async_copy(v.at[r_v], rows_v, sem).wait()

            def qbody(qq, _):
                pb = qq * _NPTS
                w16 = w_v[pl.ds(pb, _NPTS)]
                ws = [w16[p] for p in range(_NPTS)]
                for jg in range(NG):
                    acc = jnp.zeros((16,), jnp.float32)
                    for p in range(_NPTS):
                        acc = acc + ws[p] * rows_v[pb + p, pl.ds(jg * 16, 16)]
                    out_v[qq, pl.ds(jg * 16, 16)] = acc
                return 0

            lax.fori_loop(0, CQ, qbody, 0)
            pltpu.sync_copy(out_v, att.at[pl.ds(qbase, CQ)])
            return 0

        lax.fori_loop(0, NITER, chunk, 0)

    def sc_fn(vflat, sflat, w, v):
        lut = sc_scatter(vflat)
        return sc_gather(vflat, sflat, w, v, lut)

    return sc_fn, Np, Nv, DHW


def _make_stage_c(lvl):
    c = _DCHL[lvl]
    n = _NVOX[lvl]
    bn = 1000

    def body(att, q, outWt, outb, lnw, lnb, l1Wt, l1b, l2Wt, l2b, ln3w, ln3b,
             y_o):
        a = att[...]
        out = jnp.dot(a, outWt[...], preferred_element_type=jnp.float32) + outb[...]
        tgt = q[...] + out
        m = jnp.mean(tgt, axis=-1, keepdims=True)
        dev = tgt - m
        var = jnp.mean(dev * dev, axis=-1, keepdims=True)
        t = dev * lax.rsqrt(var + 1e-5) * lnw[...] + lnb[...]
        h1 = jnp.maximum(
            jnp.dot(t, l1Wt[...], preferred_element_type=jnp.float32) + l1b[...],
            0.0)
        y = jnp.dot(h1, l2Wt[...], preferred_element_type=jnp.float32) + l2b[...]
        s = t + y
        m2 = jnp.mean(s, axis=-1, keepdims=True)
        dev2 = s - m2
        var2 = jnp.mean(dev2 * dev2, axis=-1, keepdims=True)
        y_o[...] = dev2 * lax.rsqrt(var2 + 1e-5) * ln3w[...] + ln3b[...]

    grid = (n // bn,)

    def blk(i):
        return (i, 0)

    def wblk(i):
        return (0, 0)

    in_specs = [
        pl.BlockSpec((bn, c), blk),
        pl.BlockSpec((bn, c), blk),
        pl.BlockSpec((c, c), wblk),
        pl.BlockSpec((1, c), wblk),
        pl.BlockSpec((1, c), wblk),
        pl.BlockSpec((1, c), wblk),
        pl.BlockSpec((c, _DFF), wblk),
        pl.BlockSpec((1, _DFF), wblk),
        pl.BlockSpec((_DFF, c), wblk),
        pl.BlockSpec((1, c), wblk),
        pl.BlockSpec((1, c), wblk),
        pl.BlockSpec((1, c), wblk),
    ]
    out_specs = pl.BlockSpec((bn, c), blk)
    out_shape = jax.ShapeDtypeStruct((n, c), jnp.float32)
    return pl.pallas_call(body, grid=grid, in_specs=in_specs,
                          out_specs=out_specs, out_shape=out_shape)


_STAGE_A = [_make_stage_a(i) for i in range(4)]
_SC = functools.lru_cache(maxsize=None)(_make_sc)
_STAGE_C = [_make_stage_c(i) for i in range(4)]


def kernel(features_0, indices_0, features_1, indices_1, features_2,
           indices_2, features_3, indices_3, params):
    feats = [features_0, features_1, features_2, features_3]
    idxs = [indices_0, indices_1, indices_2, indices_3]
    outs = []
    for lvl in range(4):
        p = params[lvl]
        n = _NVOX[lvl]
        sc_fn, Np, Nv, DHW = _SC(lvl)
        q, v, sflat, w2, vflatb = _STAGE_A[lvl](
            feats[lvl], idxs[lvl],
            p['in_W'].T, p['in_b'][None], p['gn_w'][None], p['gn_b'][None],
            p['q_W'].T, p['q_b'][None], p['v_W'].T, p['v_b'][None],
            p['off_W'].T, p['off_b'][None], p['aw_W'].T, p['aw_b'][None])
        vflat = jnp.pad(vflatb[:, 0], (0, Nv - n), constant_values=DHW)
        sflatp = jnp.pad(sflat.reshape(n * _NPTS), (0, (Np - n) * _NPTS))
        w2p = jnp.pad(w2.reshape(n * _NPTS), (0, (Np - n) * _NPTS))
        att = sc_fn(vflat, sflatp, w2p, v)
        y = _STAGE_C[lvl](
            att[:n], q,
            p['out_W'].T, p['out_b'][None], p['ln_w'][None], p['ln_b'][None],
            p['l1_W'].T, p['l1_b'][None], p['l2_W'].T, p['l2_b'][None],
            p['ln3_w'][None], p['ln3_b'][None])
        outs.append(y)
    return tuple(outs)
